# baseline (speedup 1.0000x reference)
import functools

import jax
import jax.numpy as jnp
from jax import lax
from jax.experimental import pallas as pl
from jax.experimental.pallas import tpu as pltpu
from jax.experimental.pallas import tpu_sc as plsc

NC = 2
NS = 16
NW = NC * NS

G = 128


def _make_gather(F, BT, D):
  DT, DS = D // 8, 8
  nunit = F * BT
  upw = nunit // NW
  npair = upw // 2

  mesh = plsc.VectorSubcoreMesh(
      core_axis_name="c", subcore_axis_name="s", num_cores=NC,
      num_subcores=NS)

  @functools.partial(
      pl.kernel,
      out_type=jax.ShapeDtypeStruct((F, DT, BT, DS, G), jnp.float32),
      mesh=mesh,
      scratch_types=[
          pltpu.VMEM((upw, G), jnp.int32),
          pltpu.VMEM((upw, G), jnp.int32),
          pltpu.VMEM((2, G, 128), jnp.float32),
          pltpu.VMEM((2, DT, DS, G), jnp.float32),
          pltpu.SemaphoreType.DMA,
          pltpu.SemaphoreType.DMA,
          pltpu.SemaphoreType.DMA,
          pltpu.SemaphoreType.DMA,
      ],
      compiler_params=pltpu.CompilerParams(
          use_tc_tiling_on_sc=False, needs_layout_passes=False),
  )
  def gather_kernel(table_hbm, idx_hbm, out_hbm, idx_v, idx4_v, sg, tps,
                    gsem0, gsem1, osem0, osem1):
    wid = lax.axis_index("s") * NC + lax.axis_index("c")
    u0 = wid * upw
    pltpu.sync_copy(idx_hbm.at[pl.ds(u0, upw)], idx_v)

    def shift_body(r):
      for q in range(G // 16):
        idx4_v[r, pl.ds(16 * q, 16)] = (
            idx_v[r, pl.ds(16 * q, 16)] >> 2)

    pl.loop(0, upw)(shift_body)

    def fire(u, par, gsem):
      return pltpu.async_copy(
          table_hbm.at[idx4_v.at[u]], sg.at[par], gsem)

    def extract(u, par):
      rows = []
      offs = []
      for q in range(G // 16):
        iv = idx_v[u, pl.ds(16 * q, 16)]
        rows.append(lax.iota(jnp.int32, 16) + 16 * q)
        offs.append((iv & 3) << 5)

      def d_body(d):
        dt = d // DS
        ds_ = lax.rem(d, DS)
        for q in range(G // 16):
          v = plsc.load_gather(sg.at[par], [rows[q], offs[q] + d])
          tps[par, dt, ds_, pl.ds(16 * q, 16)] = v

      pl.loop(0, D)(d_body)

    def half(p, par, gsem, osem):
      u = 2 * p + par
      gu = u0 + u
      f = gu // BT
      bt = lax.rem(gu, BT)
      pltpu.make_async_copy(
          table_hbm.at[idx4_v.at[u]], sg.at[par], gsem).wait()
      @pl.when(p > 0)
      def _():
        pltpu.make_async_copy(
            tps.at[par], out_hbm.at[0, :, 0, :, :], osem).wait()
      extract(u, par)
      pltpu.async_copy(tps.at[par], out_hbm.at[f, :, bt, :, :], osem)
      @pl.when(p < npair - 1)
      def _():
        fire(u + 2, par, gsem)

    fire(0, 0, gsem0)
    fire(1, 1, gsem1)

    def pair_body(p):
      half(p, 0, gsem0, osem0)
      half(p, 1, gsem1, osem1)

    pl.loop(0, npair)(pair_body)

    pltpu.make_async_copy(
        tps.at[0], out_hbm.at[0, :, 0, :, :], osem0).wait()
    pltpu.make_async_copy(
        tps.at[1], out_hbm.at[0, :, 0, :, :], osem1).wait()

  return gather_kernel


def kernel(x, indices):
  V, D = x.shape
  B, F = indices.shape
  x4 = x.reshape(V * D // 128, 128)
  idxt = indices.T.reshape(F * B // G, G)
  out5 = _make_gather(F, B // G, D)(x4, idxt)
  return out5.transpose(2, 4, 0, 1, 3).reshape(B, F, D)


# speedup vs baseline: 1.0083x; 1.0083x over previous
import functools

import jax
import jax.numpy as jnp
from jax import lax
from jax.experimental import pallas as pl
from jax.experimental.pallas import tpu as pltpu
from jax.experimental.pallas import tpu_sc as plsc

NC = 2
NS = 16
NW = NC * NS

G = 128


def _make_gather(F, BT, D):
  DT, DS = D // 8, 8
  nunit = F * BT
  upw = nunit // NW
  npair = upw // 2

  mesh = plsc.VectorSubcoreMesh(
      core_axis_name="c", subcore_axis_name="s", num_cores=NC,
      num_subcores=NS)

  @functools.partial(
      pl.kernel,
      out_type=jax.ShapeDtypeStruct((F, DT, BT, DS, G), jnp.float32),
      mesh=mesh,
      scratch_types=[
          pltpu.VMEM((upw, G), jnp.int32),
          pltpu.VMEM((2, G, D), jnp.float32),
          pltpu.VMEM((2, DT, DS, G), jnp.float32),
          pltpu.SemaphoreType.DMA,
          pltpu.SemaphoreType.DMA,
          pltpu.SemaphoreType.DMA,
          pltpu.SemaphoreType.DMA,
      ],
      compiler_params=pltpu.CompilerParams(
          use_tc_tiling_on_sc=False, needs_layout_passes=False),
  )
  def gather_kernel(table_hbm, idx_hbm, out_hbm, idx_v, sg, tps,
                    gsem0, gsem1, osem0, osem1):
    wid = lax.axis_index("s") * NC + lax.axis_index("c")
    u0 = wid * upw
    pltpu.sync_copy(idx_hbm.at[pl.ds(u0, upw)], idx_v)

    def fire(u, par, gsem):
      return pltpu.async_copy(
          table_hbm.at[idx_v.at[u]], sg.at[par], gsem)

    def extract(par):
      rows = [lax.iota(jnp.int32, 16) + 16 * q for q in range(G // 16)]

      def d_body(d):
        dt = d // DS
        ds_ = lax.rem(d, DS)
        dvec = jnp.zeros((16,), jnp.int32) + d
        for q in range(G // 16):
          v = plsc.load_gather(sg.at[par], [rows[q], dvec])
          tps[par, dt, ds_, pl.ds(16 * q, 16)] = v

      pl.loop(0, D)(d_body)

    def half(p, par, gsem, osem):
      u = 2 * p + par
      gu = u0 + u
      f = gu // BT
      bt = lax.rem(gu, BT)
      pltpu.make_async_copy(
          table_hbm.at[idx_v.at[u]], sg.at[par], gsem).wait()
      @pl.when(p > 0)
      def _():
        pltpu.make_async_copy(
            tps.at[par], out_hbm.at[0, :, 0, :, :], osem).wait()
      extract(par)
      pltpu.async_copy(tps.at[par], out_hbm.at[f, :, bt, :, :], osem)
      @pl.when(p < npair - 1)
      def _():
        fire(u + 2, par, gsem)

    fire(0, 0, gsem0)
    fire(1, 1, gsem1)

    def pair_body(p):
      half(p, 0, gsem0, osem0)
      half(p, 1, gsem1, osem1)

    pl.loop(0, npair)(pair_body)

    pltpu.make_async_copy(
        tps.at[0], out_hbm.at[0, :, 0, :, :], osem0).wait()
    pltpu.make_async_copy(
        tps.at[1], out_hbm.at[0, :, 0, :, :], osem1).wait()

  return gather_kernel


def kernel(x, indices):
  V, D = x.shape
  B, F = indices.shape
  idxt = indices.T.reshape(F * B // G, G)
  out5 = _make_gather(F, B // G, D)(x, idxt)
  return out5.transpose(2, 4, 0, 1, 3).reshape(B, F, D)


# speedup vs baseline: 1.0529x; 1.0442x over previous
import functools

import jax
import jax.numpy as jnp
from jax import lax
from jax.experimental import pallas as pl
from jax.experimental.pallas import tpu as pltpu
from jax.experimental.pallas import tpu_sc as plsc

NC = 2
NS = 16
NW = NC * NS

G = 128
KPG = 13


def _make_gather(V, D, B):
  assert B % (G * NW) == 0
  ngrp = B // G
  grp_per_w = ngrp // NW
  assert grp_per_w % (2 * KPG) == 0
  npair = grp_per_w // (2 * KPG)

  mesh = plsc.VectorSubcoreMesh(
      core_axis_name="c", subcore_axis_name="s", num_cores=NC,
      num_subcores=NS)

  @functools.partial(
      pl.kernel,
      out_type=jax.ShapeDtypeStruct((ngrp, G, D), jnp.float32),
      mesh=mesh,
      scratch_types=[
          pltpu.VMEM((grp_per_w, G), jnp.int32),
          pltpu.VMEM((2, KPG, G, D), jnp.float32),
          pltpu.SemaphoreType.DMA,
          pltpu.SemaphoreType.DMA,
          pltpu.SemaphoreType.DMA,
      ],
      compiler_params=pltpu.CompilerParams(use_tc_tiling_on_sc=False),
  )
  def gather_kernel(table_hbm, idx_hbm, out_hbm, idx_v, rows_v, gsem,
                    osem0, osem1):
    wid = lax.axis_index("s") * NC + lax.axis_index("c")
    g0 = wid * grp_per_w
    pltpu.sync_copy(idx_hbm.at[pl.ds(g0, grp_per_w)], idx_v)

    def do_chunk(c, buf, osem, first):
      @pl.when(jnp.logical_not(first))
      def _():
        pltpu.make_async_copy(
            rows_v.at[buf], out_hbm.at[pl.ds(g0, KPG)], osem).wait()
      descs = [
          pltpu.async_copy(
              table_hbm.at[idx_v.at[c * KPG + j]], rows_v.at[buf, j], gsem)
          for j in range(KPG)
      ]
      for d in descs:
        d.wait()
      pltpu.async_copy(
          rows_v.at[buf], out_hbm.at[pl.ds(g0 + c * KPG, KPG)], osem)

    def pair_body(p):
      do_chunk(2 * p, 0, osem0, p == 0)
      do_chunk(2 * p + 1, 1, osem1, p == 0)

    pl.loop(0, npair)(pair_body)

    pltpu.make_async_copy(
        rows_v.at[0], out_hbm.at[pl.ds(g0, KPG)], osem0).wait()
    pltpu.make_async_copy(
        rows_v.at[1], out_hbm.at[pl.ds(g0, KPG)], osem1).wait()

  return gather_kernel


def kernel(x, indices):
  V, D = x.shape
  B = indices.size
  idx2d = indices.reshape(B // G, G)
  out = _make_gather(V, D, B)(x, idx2d)
  return out.reshape(indices.shape + (D,))
